# Initial kernel scaffold; baseline (speedup 1.0000x reference)
import jax
import jax.numpy as jnp
from jax.experimental import pallas as pl


def kernel(X_in, A_in, E_in, ecc_w1, ecc_b1, ecc_w2, ecc_b2, ecc_root, ecc_bias, gat_w, gat_bias, gat_attn_src, gat_attn_dst, fc_w, fc_b):
    raise NotImplementedError("write your pallas kernel here")


# speedup vs baseline: 3.5135x; 3.5135x over previous
import functools
import jax
import jax.numpy as jnp
from jax import lax
from jax.experimental import pallas as pl
from jax.experimental.pallas import tpu as pltpu
from jax.experimental.pallas import tpu_sc as plsc

N = 10000
E = 320000
D = 128
F = 64
KH = 32
KTOT = KH + 1
HW = 528
YW = 2176
AW = 128
NSUB = 16
NCORE = 2
NTILE = NCORE * NSUB
EPT = E // NTILE
NP = 10240
RPT = NP // NSUB
CH1 = 8
ZR = 32
CH2 = 80


def _h_body(e_ref, w1_ref, b1_ref, rep_ref, out_ref):
    h = jnp.dot(e_ref[...], w1_ref[...], preferred_element_type=jnp.float32)
    h = jnp.maximum(h + b1_ref[...], 0.0)
    b = h.shape[0]
    hx = jnp.concatenate([h, jnp.ones((b, 1), jnp.float32)], axis=1)
    out_ref[...] = jnp.dot(hx, rep_ref[...], preferred_element_type=jnp.float32)


def _make_h(e_in, w1, b1):
    blk = 4000
    grid = E // blk
    rep = jnp.repeat(jnp.eye(KTOT, dtype=jnp.float32), 16, axis=1)
    return pl.pallas_call(
        _h_body,
        grid=(grid,),
        in_specs=[
            pl.BlockSpec((blk, 4), lambda i: (i, 0)),
            pl.BlockSpec((4, KH), lambda i: (0, 0)),
            pl.BlockSpec((1, KH), lambda i: (0, 0)),
            pl.BlockSpec((KTOT, HW), lambda i: (0, 0)),
        ],
        out_specs=pl.BlockSpec((blk, HW), lambda i: (i, 0)),
        out_shape=jax.ShapeDtypeStruct((E, HW), jnp.float32),
    )(e_in, w1, b1.reshape(1, KH), rep)


def _y_body(x_ref, w_ref, out_ref):
    out_ref[...] = jnp.dot(x_ref[...], w_ref[...],
                           preferred_element_type=jnp.float32)


def _make_y(x, w2ext):
    blk = 1000
    grid = N // blk
    return pl.pallas_call(
        _y_body,
        grid=(grid,),
        in_specs=[
            pl.BlockSpec((blk, D), lambda i: (i, 0)),
            pl.BlockSpec((D, YW), lambda i: (0, 0)),
        ],
        out_specs=pl.BlockSpec((blk, YW), lambda i: (i, 0)),
        out_shape=jax.ShapeDtypeStruct((N, YW), jnp.float32),
    )(x, w2ext)


def _msg_sc(y_hbm, h_hbm, src_hbm, dst_hbm, out_hbm,
            src_v, dst_v, rows_v, h_v, msg_v, zbuf_v, accum_sh, sem):
    cid = lax.axis_index("c")
    sid = lax.axis_index("s")
    wid = cid * NSUB + sid

    def _zrow(r, _):
        for c in range(AW // 16):
            zbuf_v[r, pl.ds(c * 16, 16)] = jnp.zeros((16,), jnp.float32)
        return 0
    lax.fori_loop(0, ZR, _zrow, 0)
    for r in range(RPT // ZR):
        pltpu.sync_copy(zbuf_v, accum_sh.at[pl.ds(sid * RPT + r * ZR, ZR)])
    plsc.subcore_barrier()

    base0 = wid * EPT

    def _chunk(ci, _):
        b = base0 + ci * CH1
        pltpu.sync_copy(src_hbm.at[pl.ds(b, CH1)], src_v)
        pltpu.sync_copy(dst_hbm.at[pl.ds(b, CH1)], dst_v)
        pltpu.sync_copy(h_hbm.at[pl.ds(b, CH1)], h_v)
        pltpu.async_copy(y_hbm.at[src_v], rows_v, sem).wait()

        def _edge(e, _):
            acc = [jnp.zeros((16,), jnp.float32) for _ in range(F // 16)]
            for k in range(KTOT):
                hk = h_v[e, pl.ds(k * 16, 16)]
                for j in range(F // 16):
                    acc[j] = acc[j] + hk * rows_v[e, pl.ds(k * F + j * 16, 16)]
            for j in range(F // 16):
                msg_v[e, pl.ds(j * 16, 16)] = acc[j]
            msg_v[e, pl.ds(F, 16)] = jnp.ones((16,), jnp.float32)
            for j in range(F // 16 + 1, AW // 16):
                msg_v[e, pl.ds(j * 16, 16)] = jnp.zeros((16,), jnp.float32)
            return 0
        lax.fori_loop(0, CH1, _edge, 0)
        pltpu.sync_copy(msg_v, accum_sh.at[dst_v], add=True)
        return 0
    lax.fori_loop(0, EPT // CH1, _chunk, 0)
    plsc.subcore_barrier()

    for r in range(RPT // ZR):
        pltpu.sync_copy(accum_sh.at[pl.ds(sid * RPT + r * ZR, ZR)], zbuf_v)
        pltpu.sync_copy(zbuf_v, out_hbm.at[cid, pl.ds(sid * RPT + r * ZR, ZR)])


def _run_msg(y, h, src, dst):
    mesh = plsc.VectorSubcoreMesh(core_axis_name="c", subcore_axis_name="s")
    fn = pl.kernel(
        _msg_sc, mesh=mesh,
        out_type=jax.ShapeDtypeStruct((NCORE, NP, AW), jnp.float32),
        scratch_types=[
            pltpu.VMEM((CH1,), jnp.int32),
            pltpu.VMEM((CH1,), jnp.int32),
            pltpu.VMEM((CH1, YW), jnp.float32),
            pltpu.VMEM((CH1, HW), jnp.float32),
            pltpu.VMEM((CH1, AW), jnp.float32),
            pltpu.VMEM((ZR, AW), jnp.float32),
            pltpu.VMEM_SHARED((NP, AW), jnp.float32),
            pltpu.SemaphoreType.DMA,
        ],
    )
    return fn(y, h, src, dst)


def _node_body(msum_ref, x_ref, root_ref, eb_ref, gw_ref, asr_ref, adr_ref,
               tsrc_ref, tdst_ref, maxes_ref):
    i = pl.program_id(0)
    a = msum_ref[0] + msum_ref[1]
    deg = jnp.maximum(a[:, F:F + 1], 1.0)
    agg = a[:, :F] / deg
    x = jnp.maximum(
        agg + jnp.dot(x_ref[...], root_ref[...],
                      preferred_element_type=jnp.float32) + eb_ref[...], 0.0)
    xw = jnp.dot(x, gw_ref[...], preferred_element_type=jnp.float32)
    a_s = jnp.sum(xw * asr_ref[...], axis=1, keepdims=True)
    a_d = jnp.sum(xw * adr_ref[...], axis=1, keepdims=True)
    b = xw.shape[0]
    tsrc_ref[...] = jnp.concatenate(
        [xw, jnp.broadcast_to(a_s, (b, 16)),
         jnp.zeros((b, 128 - F - 16), jnp.float32)], axis=1)
    tdst_ref[...] = jnp.concatenate(
        [jnp.broadcast_to(a_d, (b, 16)),
         jnp.zeros((b, 112), jnp.float32)], axis=1)
    lane = lax.broadcasted_iota(jnp.int32, (1, 128), 1)
    cand = jnp.where(lane == 0, jnp.max(a_s),
                     jnp.where(lane == 1, jnp.max(a_d), -jnp.inf))

    @pl.when(i == 0)
    def _():
        maxes_ref[...] = jnp.full((1, 128), -jnp.inf, jnp.float32)
    maxes_ref[...] = jnp.maximum(maxes_ref[...], cand)


def _run_node(msum, x, root, eb, gw, asr, adr):
    blk = 1000
    grid = N // blk
    return pl.pallas_call(
        _node_body,
        grid=(grid,),
        in_specs=[
            pl.BlockSpec((NCORE, blk, AW), lambda i: (0, i, 0)),
            pl.BlockSpec((blk, D), lambda i: (i, 0)),
            pl.BlockSpec((D, F), lambda i: (0, 0)),
            pl.BlockSpec((1, F), lambda i: (0, 0)),
            pl.BlockSpec((F, F), lambda i: (0, 0)),
            pl.BlockSpec((1, F), lambda i: (0, 0)),
            pl.BlockSpec((1, F), lambda i: (0, 0)),
        ],
        out_specs=[
            pl.BlockSpec((blk, 128), lambda i: (i, 0)),
            pl.BlockSpec((blk, 128), lambda i: (i, 0)),
            pl.BlockSpec((1, 128), lambda i: (0, 0)),
        ],
        out_shape=[
            jax.ShapeDtypeStruct((N, 128), jnp.float32),
            jax.ShapeDtypeStruct((N, 128), jnp.float32),
            jax.ShapeDtypeStruct((1, 128), jnp.float32),
        ],
    )(msum, x, root, eb.reshape(1, F), gw, asr.reshape(1, F),
      adr.reshape(1, F))


def _gat_sc(tsrc_hbm, tdst_hbm, src_hbm, dst_hbm, m_hbm, out_hbm,
            src_v, dst_v, rx_v, rd_v, o_v, m_v, zbuf_v, accum_sh, sem):
    cid = lax.axis_index("c")
    sid = lax.axis_index("s")
    wid = cid * NSUB + sid

    def _zrow(r, _):
        for c in range(AW // 16):
            zbuf_v[r, pl.ds(c * 16, 16)] = jnp.zeros((16,), jnp.float32)
        return 0
    lax.fori_loop(0, ZR, _zrow, 0)
    for r in range(RPT // ZR):
        pltpu.sync_copy(zbuf_v, accum_sh.at[pl.ds(sid * RPT + r * ZR, ZR)])
    pltpu.sync_copy(m_hbm, m_v)
    plsc.subcore_barrier()

    base0 = wid * EPT

    def _chunk(ci, _):
        b = base0 + ci * CH2
        pltpu.sync_copy(src_hbm.at[pl.ds(b, CH2)], src_v)
        pltpu.sync_copy(dst_hbm.at[pl.ds(b, CH2)], dst_v)
        cp1 = pltpu.async_copy(tsrc_hbm.at[src_v], rx_v, sem)
        cp2 = pltpu.async_copy(tdst_hbm.at[dst_v], rd_v, sem)
        cp1.wait()
        cp2.wait()
        mvec = m_v[...]

        def _edge(e, _):
            s = rx_v[e, pl.ds(F, 16)] + rd_v[e, pl.ds(0, 16)]
            ee = jnp.maximum(s, 0.2 * s)
            w = jnp.exp(ee - mvec)
            for j in range(F // 16):
                o_v[e, pl.ds(j * 16, 16)] = w * rx_v[e, pl.ds(j * 16, 16)]
            o_v[e, pl.ds(F, 16)] = w
            for j in range(F // 16 + 1, AW // 16):
                o_v[e, pl.ds(j * 16, 16)] = jnp.zeros((16,), jnp.float32)
            return 0
        lax.fori_loop(0, CH2, _edge, 0)
        pltpu.sync_copy(o_v, accum_sh.at[dst_v], add=True)
        return 0
    lax.fori_loop(0, EPT // CH2, _chunk, 0)
    plsc.subcore_barrier()

    for r in range(RPT // ZR):
        pltpu.sync_copy(accum_sh.at[pl.ds(sid * RPT + r * ZR, ZR)], zbuf_v)
        pltpu.sync_copy(zbuf_v, out_hbm.at[cid, pl.ds(sid * RPT + r * ZR, ZR)])


def _run_gat(tsrc, tdst, src, dst, marr):
    mesh = plsc.VectorSubcoreMesh(core_axis_name="c", subcore_axis_name="s")
    fn = pl.kernel(
        _gat_sc, mesh=mesh,
        out_type=jax.ShapeDtypeStruct((NCORE, NP, AW), jnp.float32),
        scratch_types=[
            pltpu.VMEM((CH2,), jnp.int32),
            pltpu.VMEM((CH2,), jnp.int32),
            pltpu.VMEM((CH2, 128), jnp.float32),
            pltpu.VMEM((CH2, 128), jnp.float32),
            pltpu.VMEM((CH2, AW), jnp.float32),
            pltpu.VMEM((16,), jnp.float32),
            pltpu.VMEM((ZR, AW), jnp.float32),
            pltpu.VMEM_SHARED((NP, AW), jnp.float32),
            pltpu.SemaphoreType.DMA,
        ],
    )
    return fn(tsrc, tdst, src, dst, marr)


def _final_body(acc_ref, gb_ref, fw_ref, fb_ref, out_ref, psum_ref):
    i = pl.program_id(0)
    a = acc_ref[0] + acc_ref[1]
    den = jnp.maximum(a[:, F:F + 1], 1e-9)
    x2 = jnp.maximum(a[:, :F] / den + gb_ref[...], 0.0)

    @pl.when(i == 0)
    def _():
        psum_ref[...] = jnp.zeros((8, F), jnp.float32)
    psum_ref[0:1, :] = psum_ref[0:1, :] + jnp.sum(x2, axis=0, keepdims=True)
    pooled = psum_ref[0:1, :] * (1.0 / N)
    out_ref[...] = jnp.maximum(
        jnp.dot(pooled, fw_ref[...], preferred_element_type=jnp.float32)
        + fb_ref[...], 0.0)


def _run_final(acc, gb, fw, fb):
    blk = 1000
    grid = N // blk
    return pl.pallas_call(
        _final_body,
        grid=(grid,),
        in_specs=[
            pl.BlockSpec((NCORE, blk, AW), lambda i: (0, i, 0)),
            pl.BlockSpec((1, F), lambda i: (0, 0)),
            pl.BlockSpec((F, 32), lambda i: (0, 0)),
            pl.BlockSpec((1, 32), lambda i: (0, 0)),
        ],
        out_specs=pl.BlockSpec((1, 32), lambda i: (0, 0)),
        out_shape=jax.ShapeDtypeStruct((1, 32), jnp.float32),
        scratch_shapes=[pltpu.VMEM((8, F), jnp.float32)],
    )(acc, gb.reshape(1, F), fw, fb.reshape(1, 32))


def kernel(X_in, A_in, E_in, ecc_w1, ecc_b1, ecc_w2, ecc_b2, ecc_root,
           ecc_bias, gat_w, gat_bias, gat_attn_src, gat_attn_dst, fc_w, fc_b):
    src = A_in[0].astype(jnp.int32)
    dst = A_in[1].astype(jnp.int32)

    w2r = ecc_w2.reshape(KH, D, F).transpose(1, 0, 2).reshape(D, KH * F)
    w2ext = jnp.concatenate(
        [w2r, ecc_b2.reshape(D, F),
         jnp.zeros((D, YW - KTOT * F), jnp.float32)], axis=1)

    h = _make_h(E_in, ecc_w1, ecc_b1)
    y = _make_y(X_in, w2ext)
    msum = _run_msg(y, h, src, dst)
    tsrc, tdst, maxes = _run_node(msum, X_in, ecc_root, ecc_bias,
                                  gat_w, gat_attn_src, gat_attn_dst)
    m = jnp.maximum(maxes[0, 0] + maxes[0, 1], 0.0)
    marr = jnp.full((16,), m, jnp.float32)
    acc2 = _run_gat(tsrc, tdst, src, dst, marr)
    return _run_final(acc2, gat_bias, fc_w, fc_b)


# speedup vs baseline: 4.2813x; 1.2185x over previous
import functools
import jax
import jax.numpy as jnp
from jax import lax
from jax.experimental import pallas as pl
from jax.experimental.pallas import tpu as pltpu
from jax.experimental.pallas import tpu_sc as plsc

N = 10000
E = 320000
D = 128
F = 64
KH = 32
KTOT = KH + 1
HW = 528
YW = 2176
AW = 128
NSUB = 16
NCORE = 2
NTILE = NCORE * NSUB
EPT = E // NTILE
NP = 10240
RPT = NP // NSUB
CH1 = 8
ZR = 16
CH2 = 80


def _h_body(e_ref, w1_ref, b1_ref, rep_ref, out_ref):
    h = jnp.dot(e_ref[...], w1_ref[...], preferred_element_type=jnp.float32)
    h = jnp.maximum(h + b1_ref[...], 0.0)
    b = h.shape[0]
    hx = jnp.concatenate([h, jnp.ones((b, 1), jnp.float32)], axis=1)
    out_ref[...] = jnp.dot(hx, rep_ref[...], preferred_element_type=jnp.float32)


def _make_h(e_in, w1, b1):
    blk = 4000
    grid = E // blk
    rep = jnp.repeat(jnp.eye(KTOT, dtype=jnp.float32), 16, axis=1)
    return pl.pallas_call(
        _h_body,
        grid=(grid,),
        in_specs=[
            pl.BlockSpec((blk, 4), lambda i: (i, 0)),
            pl.BlockSpec((4, KH), lambda i: (0, 0)),
            pl.BlockSpec((1, KH), lambda i: (0, 0)),
            pl.BlockSpec((KTOT, HW), lambda i: (0, 0)),
        ],
        out_specs=pl.BlockSpec((blk, HW), lambda i: (i, 0)),
        out_shape=jax.ShapeDtypeStruct((E, HW), jnp.float32),
    )(e_in, w1, b1.reshape(1, KH), rep)


def _y_body(x_ref, w_ref, out_ref):
    out_ref[...] = jnp.dot(x_ref[...], w_ref[...],
                           preferred_element_type=jnp.float32)


def _make_y(x, w2ext):
    blk = 1000
    grid = N // blk
    return pl.pallas_call(
        _y_body,
        grid=(grid,),
        in_specs=[
            pl.BlockSpec((blk, D), lambda i: (i, 0)),
            pl.BlockSpec((D, YW), lambda i: (0, 0)),
        ],
        out_specs=pl.BlockSpec((blk, YW), lambda i: (i, 0)),
        out_shape=jax.ShapeDtypeStruct((N, YW), jnp.float32),
    )(x, w2ext)


def _msg_sc(y_hbm, h_hbm, src_hbm, dst_hbm, out_hbm,
            src_v, dst_v, rows_v, h_v, msg_v, zbuf_v, accum_sh, sems):
    cid = lax.axis_index("c")
    sid = lax.axis_index("s")
    wid = cid * NSUB + sid

    def _zrow(r, _):
        for c in range(AW // 16):
            zbuf_v[r, pl.ds(c * 16, 16)] = jnp.zeros((16,), jnp.float32)
        return 0
    lax.fori_loop(0, ZR, _zrow, 0)
    for r in range(RPT // ZR):
        pltpu.sync_copy(zbuf_v, accum_sh.at[pl.ds(sid * RPT + r * ZR, ZR)])
    plsc.subcore_barrier()

    base0 = wid * EPT
    nchunks = EPT // CH1

    def _issue(ci, par):
        b = base0 + ci * CH1
        pltpu.sync_copy(src_hbm.at[pl.ds(b, CH1)], src_v.at[par])
        pltpu.sync_copy(dst_hbm.at[pl.ds(b, CH1)], dst_v.at[par])
        pltpu.sync_copy(h_hbm.at[pl.ds(b, CH1)], h_v.at[par])
        pltpu.async_copy(y_hbm.at[src_v.at[par]], rows_v.at[par],
                         sems.at[par])

    _issue(0, 0)

    def _chunk(ci, _):
        par = lax.rem(ci, 2)
        pltpu.make_async_copy(y_hbm.at[pl.ds(0, CH1)], rows_v.at[par],
                              sems.at[par]).wait()

        @pl.when(ci + 1 < nchunks)
        def _():
            _issue(ci + 1, 1 - par)

        def _edge(e, _):
            acc = [jnp.zeros((16,), jnp.float32) for _ in range(F // 16)]
            for k in range(KTOT):
                hk = h_v[par, e, pl.ds(k * 16, 16)]
                for j in range(F // 16):
                    acc[j] = acc[j] + hk * rows_v[par, e,
                                                  pl.ds(k * F + j * 16, 16)]
            for j in range(F // 16):
                msg_v[e, pl.ds(j * 16, 16)] = acc[j]
            msg_v[e, pl.ds(F, 16)] = jnp.ones((16,), jnp.float32)
            for j in range(F // 16 + 1, AW // 16):
                msg_v[e, pl.ds(j * 16, 16)] = jnp.zeros((16,), jnp.float32)
            return 0
        lax.fori_loop(0, CH1, _edge, 0)
        pltpu.sync_copy(msg_v, accum_sh.at[dst_v.at[par]], add=True)
        return 0
    lax.fori_loop(0, nchunks, _chunk, 0)
    plsc.subcore_barrier()

    for r in range(RPT // ZR):
        pltpu.sync_copy(accum_sh.at[pl.ds(sid * RPT + r * ZR, ZR)], zbuf_v)
        pltpu.sync_copy(zbuf_v, out_hbm.at[cid, pl.ds(sid * RPT + r * ZR, ZR)])


def _run_msg(y, h, src, dst):
    mesh = plsc.VectorSubcoreMesh(core_axis_name="c", subcore_axis_name="s")
    fn = pl.kernel(
        _msg_sc, mesh=mesh,
        out_type=jax.ShapeDtypeStruct((NCORE, NP, AW), jnp.float32),
        scratch_types=[
            pltpu.VMEM((2, CH1), jnp.int32),
            pltpu.VMEM((2, CH1), jnp.int32),
            pltpu.VMEM((2, CH1, YW), jnp.float32),
            pltpu.VMEM((2, CH1, HW), jnp.float32),
            pltpu.VMEM((CH1, AW), jnp.float32),
            pltpu.VMEM((ZR, AW), jnp.float32),
            pltpu.VMEM_SHARED((NP, AW), jnp.float32),
            pltpu.SemaphoreType.DMA((2,)),
        ],
    )
    return fn(y, h, src, dst)


def _node_body(msum_ref, x_ref, root_ref, eb_ref, gw_ref, asr_ref, adr_ref,
               tsrc_ref, tdst_ref, maxes_ref):
    i = pl.program_id(0)
    a = msum_ref[0] + msum_ref[1]
    deg = jnp.maximum(a[:, F:F + 1], 1.0)
    agg = a[:, :F] / deg
    x = jnp.maximum(
        agg + jnp.dot(x_ref[...], root_ref[...],
                      preferred_element_type=jnp.float32) + eb_ref[...], 0.0)
    xw = jnp.dot(x, gw_ref[...], preferred_element_type=jnp.float32)
    a_s = jnp.sum(xw * asr_ref[...], axis=1, keepdims=True)
    a_d = jnp.sum(xw * adr_ref[...], axis=1, keepdims=True)
    b = xw.shape[0]
    tsrc_ref[...] = jnp.concatenate(
        [xw, jnp.broadcast_to(a_s, (b, 16)),
         jnp.zeros((b, 128 - F - 16), jnp.float32)], axis=1)
    tdst_ref[...] = jnp.concatenate(
        [jnp.broadcast_to(a_d, (b, 16)),
         jnp.zeros((b, 112), jnp.float32)], axis=1)
    lane = lax.broadcasted_iota(jnp.int32, (1, 128), 1)
    cand = jnp.where(lane == 0, jnp.max(a_s),
                     jnp.where(lane == 1, jnp.max(a_d), -jnp.inf))

    @pl.when(i == 0)
    def _():
        maxes_ref[...] = jnp.full((1, 128), -jnp.inf, jnp.float32)
    maxes_ref[...] = jnp.maximum(maxes_ref[...], cand)


def _run_node(msum, x, root, eb, gw, asr, adr):
    blk = 1000
    grid = N // blk
    return pl.pallas_call(
        _node_body,
        grid=(grid,),
        in_specs=[
            pl.BlockSpec((NCORE, blk, AW), lambda i: (0, i, 0)),
            pl.BlockSpec((blk, D), lambda i: (i, 0)),
            pl.BlockSpec((D, F), lambda i: (0, 0)),
            pl.BlockSpec((1, F), lambda i: (0, 0)),
            pl.BlockSpec((F, F), lambda i: (0, 0)),
            pl.BlockSpec((1, F), lambda i: (0, 0)),
            pl.BlockSpec((1, F), lambda i: (0, 0)),
        ],
        out_specs=[
            pl.BlockSpec((blk, 128), lambda i: (i, 0)),
            pl.BlockSpec((blk, 128), lambda i: (i, 0)),
            pl.BlockSpec((1, 128), lambda i: (0, 0)),
        ],
        out_shape=[
            jax.ShapeDtypeStruct((N, 128), jnp.float32),
            jax.ShapeDtypeStruct((N, 128), jnp.float32),
            jax.ShapeDtypeStruct((1, 128), jnp.float32),
        ],
    )(msum, x, root, eb.reshape(1, F), gw, asr.reshape(1, F),
      adr.reshape(1, F))


def _gat_sc(tsrc_hbm, tdst_hbm, src_hbm, dst_hbm, m_hbm, out_hbm,
            src_v, dst_v, rx_v, rd_v, o_v, m_v, zbuf_v, accum_sh, sem):
    cid = lax.axis_index("c")
    sid = lax.axis_index("s")
    wid = cid * NSUB + sid

    def _zrow(r, _):
        for c in range(AW // 16):
            zbuf_v[r, pl.ds(c * 16, 16)] = jnp.zeros((16,), jnp.float32)
        return 0
    lax.fori_loop(0, ZR, _zrow, 0)
    for r in range(RPT // ZR):
        pltpu.sync_copy(zbuf_v, accum_sh.at[pl.ds(sid * RPT + r * ZR, ZR)])
    pltpu.sync_copy(m_hbm, m_v)
    plsc.subcore_barrier()

    base0 = wid * EPT

    def _chunk(ci, _):
        b = base0 + ci * CH2
        pltpu.sync_copy(src_hbm.at[pl.ds(b, CH2)], src_v)
        pltpu.sync_copy(dst_hbm.at[pl.ds(b, CH2)], dst_v)
        cp1 = pltpu.async_copy(tsrc_hbm.at[src_v], rx_v, sem)
        cp2 = pltpu.async_copy(tdst_hbm.at[dst_v], rd_v, sem)
        cp1.wait()
        cp2.wait()
        mvec = m_v[...]

        def _edge(e, _):
            s = rx_v[e, pl.ds(F, 16)] + rd_v[e, pl.ds(0, 16)]
            ee = jnp.maximum(s, 0.2 * s)
            w = jnp.exp(ee - mvec)
            for j in range(F // 16):
                o_v[e, pl.ds(j * 16, 16)] = w * rx_v[e, pl.ds(j * 16, 16)]
            o_v[e, pl.ds(F, 16)] = w
            for j in range(F // 16 + 1, AW // 16):
                o_v[e, pl.ds(j * 16, 16)] = jnp.zeros((16,), jnp.float32)
            return 0
        lax.fori_loop(0, CH2, _edge, 0)
        pltpu.sync_copy(o_v, accum_sh.at[dst_v], add=True)
        return 0
    lax.fori_loop(0, EPT // CH2, _chunk, 0)
    plsc.subcore_barrier()

    for r in range(RPT // ZR):
        pltpu.sync_copy(accum_sh.at[pl.ds(sid * RPT + r * ZR, ZR)], zbuf_v)
        pltpu.sync_copy(zbuf_v, out_hbm.at[cid, pl.ds(sid * RPT + r * ZR, ZR)])


def _run_gat(tsrc, tdst, src, dst, marr):
    mesh = plsc.VectorSubcoreMesh(core_axis_name="c", subcore_axis_name="s")
    fn = pl.kernel(
        _gat_sc, mesh=mesh,
        out_type=jax.ShapeDtypeStruct((NCORE, NP, AW), jnp.float32),
        scratch_types=[
            pltpu.VMEM((CH2,), jnp.int32),
            pltpu.VMEM((CH2,), jnp.int32),
            pltpu.VMEM((CH2, 128), jnp.float32),
            pltpu.VMEM((CH2, 128), jnp.float32),
            pltpu.VMEM((CH2, AW), jnp.float32),
            pltpu.VMEM((16,), jnp.float32),
            pltpu.VMEM((ZR, AW), jnp.float32),
            pltpu.VMEM_SHARED((NP, AW), jnp.float32),
            pltpu.SemaphoreType.DMA,
        ],
    )
    return fn(tsrc, tdst, src, dst, marr)


def _final_body(acc_ref, gb_ref, fw_ref, fb_ref, out_ref, psum_ref):
    i = pl.program_id(0)
    a = acc_ref[0] + acc_ref[1]
    den = jnp.maximum(a[:, F:F + 1], 1e-9)
    x2 = jnp.maximum(a[:, :F] / den + gb_ref[...], 0.0)

    @pl.when(i == 0)
    def _():
        psum_ref[...] = jnp.zeros((8, F), jnp.float32)
    psum_ref[0:1, :] = psum_ref[0:1, :] + jnp.sum(x2, axis=0, keepdims=True)
    pooled = psum_ref[0:1, :] * (1.0 / N)
    out_ref[...] = jnp.maximum(
        jnp.dot(pooled, fw_ref[...], preferred_element_type=jnp.float32)
        + fb_ref[...], 0.0)


def _run_final(acc, gb, fw, fb):
    blk = 1000
    grid = N // blk
    return pl.pallas_call(
        _final_body,
        grid=(grid,),
        in_specs=[
            pl.BlockSpec((NCORE, blk, AW), lambda i: (0, i, 0)),
            pl.BlockSpec((1, F), lambda i: (0, 0)),
            pl.BlockSpec((F, 32), lambda i: (0, 0)),
            pl.BlockSpec((1, 32), lambda i: (0, 0)),
        ],
        out_specs=pl.BlockSpec((1, 32), lambda i: (0, 0)),
        out_shape=jax.ShapeDtypeStruct((1, 32), jnp.float32),
        scratch_shapes=[pltpu.VMEM((8, F), jnp.float32)],
    )(acc, gb.reshape(1, F), fw, fb.reshape(1, 32))


def kernel(X_in, A_in, E_in, ecc_w1, ecc_b1, ecc_w2, ecc_b2, ecc_root,
           ecc_bias, gat_w, gat_bias, gat_attn_src, gat_attn_dst, fc_w, fc_b):
    src = A_in[0].astype(jnp.int32)
    dst = A_in[1].astype(jnp.int32)

    w2r = ecc_w2.reshape(KH, D, F).transpose(1, 0, 2).reshape(D, KH * F)
    w2ext = jnp.concatenate(
        [w2r, ecc_b2.reshape(D, F),
         jnp.zeros((D, YW - KTOT * F), jnp.float32)], axis=1)

    h = _make_h(E_in, ecc_w1, ecc_b1)
    y = _make_y(X_in, w2ext)
    msum = _run_msg(y, h, src, dst)
    tsrc, tdst, maxes = _run_node(msum, X_in, ecc_root, ecc_bias,
                                  gat_w, gat_attn_src, gat_attn_dst)
    m = jnp.maximum(maxes[0, 0] + maxes[0, 1], 0.0)
    marr = jnp.full((16,), m, jnp.float32)
    acc2 = _run_gat(tsrc, tdst, src, dst, marr)
    return _run_final(acc2, gat_bias, fc_w, fc_b)


# speedup vs baseline: 6.9398x; 1.6210x over previous
import functools
import jax
import jax.numpy as jnp
from jax import lax
from jax.experimental import pallas as pl
from jax.experimental.pallas import tpu as pltpu
from jax.experimental.pallas import tpu_sc as plsc

N = 10000
E = 320000
D = 128
F = 64
KH = 32
KTOT = KH + 1
HW = 528
YW = 2176
AW = 128
NSUB = 16
NCORE = 2
NTILE = NCORE * NSUB
EPT = E // NTILE
NP = 10112
RPT = NP // NSUB
CH1 = 8
ZR = 8
IBC = 125
CH2 = 80


def _h_body(e_ref, w1_ref, b1_ref, rep_ref, out_ref):
    h = jnp.dot(e_ref[...], w1_ref[...], preferred_element_type=jnp.float32)
    h = jnp.maximum(h + b1_ref[...], 0.0)
    b = h.shape[0]
    hx = jnp.concatenate([h, jnp.ones((b, 1), jnp.float32)], axis=1)
    out_ref[...] = jnp.dot(hx, rep_ref[...], preferred_element_type=jnp.float32)


def _make_h(e_in, w1, b1):
    blk = 4000
    grid = E // blk
    rep = jnp.repeat(jnp.eye(KTOT, dtype=jnp.float32), 16, axis=1)
    return pl.pallas_call(
        _h_body,
        grid=(grid,),
        in_specs=[
            pl.BlockSpec((blk, 4), lambda i: (i, 0)),
            pl.BlockSpec((4, KH), lambda i: (0, 0)),
            pl.BlockSpec((1, KH), lambda i: (0, 0)),
            pl.BlockSpec((KTOT, HW), lambda i: (0, 0)),
        ],
        out_specs=pl.BlockSpec((blk, HW), lambda i: (i, 0)),
        out_shape=jax.ShapeDtypeStruct((E, HW), jnp.float32),
    )(e_in, w1, b1.reshape(1, KH), rep)


def _y_body(x_ref, w_ref, out_ref):
    out_ref[...] = jnp.dot(x_ref[...], w_ref[...],
                           preferred_element_type=jnp.float32)


def _make_y(x, w2ext):
    blk = 1000
    grid = N // blk
    return pl.pallas_call(
        _y_body,
        grid=(grid,),
        in_specs=[
            pl.BlockSpec((blk, D), lambda i: (i, 0)),
            pl.BlockSpec((D, YW), lambda i: (0, 0)),
        ],
        out_specs=pl.BlockSpec((blk, YW), lambda i: (i, 0)),
        out_shape=jax.ShapeDtypeStruct((N, YW), jnp.float32),
    )(x, w2ext)


def _msg_sc(y_hbm, h_hbm, src_hbm, dst_hbm, out_hbm,
            srcb_v, dst_v, rows_v, h_v, msg_v, accum_sh,
            sems_g, sems_m, sems_s):
    cid = lax.axis_index("c")
    sid = lax.axis_index("s")
    wid = cid * NSUB + sid

    def _zrow(r, _):
        for c in range(AW // 16):
            msg_v[0, r, pl.ds(c * 16, 16)] = jnp.zeros((16,), jnp.float32)
        return 0
    lax.fori_loop(0, ZR, _zrow, 0)
    for r in range(RPT // ZR):
        pltpu.sync_copy(msg_v.at[0], accum_sh.at[pl.ds(sid * RPT + r * ZR, ZR)])
    plsc.subcore_barrier()

    base0 = wid * EPT
    nchunks = EPT // CH1

    def _issue(ci):
        par = lax.rem(ci, 2)
        s3 = lax.rem(ci, 3)
        b = base0 + ci * CH1

        @pl.when(jnp.logical_and(lax.rem(ci, IBC) == 0, True))
        def _():
            pltpu.sync_copy(src_hbm.at[pl.ds(b, IBC * CH1)], srcb_v)

        @pl.when(ci >= 3)
        def _():
            pltpu.make_async_copy(msg_v.at[s3], accum_sh.at[pl.ds(0, CH1)],
                                  sems_s.at[s3]).wait()
        pltpu.async_copy(dst_hbm.at[pl.ds(b, CH1)], dst_v.at[s3],
                         sems_m.at[par])
        pltpu.async_copy(h_hbm.at[pl.ds(b, CH1)], h_v.at[par],
                         sems_m.at[par])
        off = lax.rem(ci, IBC) * CH1
        pltpu.async_copy(y_hbm.at[srcb_v.at[pl.ds(off, CH1)]],
                         rows_v.at[par], sems_g.at[par])

    _issue(0)

    def _chunk(ci, _):
        par = lax.rem(ci, 2)
        s3 = lax.rem(ci, 3)
        pltpu.make_async_copy(y_hbm.at[pl.ds(0, CH1)], rows_v.at[par],
                              sems_g.at[par]).wait()
        pltpu.make_async_copy(dst_hbm.at[pl.ds(0, CH1)], dst_v.at[s3],
                              sems_m.at[par]).wait()
        pltpu.make_async_copy(h_hbm.at[pl.ds(0, CH1)], h_v.at[par],
                              sems_m.at[par]).wait()

        @pl.when(ci + 1 < nchunks)
        def _():
            _issue(ci + 1)

        def _edge(e, _):
            acc = [jnp.zeros((16,), jnp.float32) for _ in range(F // 16)]
            for k in range(KTOT):
                hk = h_v[par, e, pl.ds(k * 16, 16)]
                for j in range(F // 16):
                    acc[j] = acc[j] + hk * rows_v[par, e,
                                                  pl.ds(k * F + j * 16, 16)]
            for j in range(F // 16):
                msg_v[s3, e, pl.ds(j * 16, 16)] = acc[j]
            msg_v[s3, e, pl.ds(F, 16)] = jnp.ones((16,), jnp.float32)
            for j in range(F // 16 + 1, AW // 16):
                msg_v[s3, e, pl.ds(j * 16, 16)] = jnp.zeros((16,), jnp.float32)
            return 0
        lax.fori_loop(0, CH1, _edge, 0)
        pltpu.async_copy(msg_v.at[s3], accum_sh.at[dst_v.at[s3]],
                         sems_s.at[s3], add=True)
        return 0
    lax.fori_loop(0, nchunks, _chunk, 0)
    for sl in range(3):
        pltpu.make_async_copy(msg_v.at[sl], accum_sh.at[pl.ds(0, CH1)],
                              sems_s.at[sl]).wait()
    plsc.subcore_barrier()

    pltpu.sync_copy(accum_sh.at[pl.ds(sid * RPT, RPT)],
                    out_hbm.at[cid, pl.ds(sid * RPT, RPT)])


def _run_msg(y, h, src, dst):
    mesh = plsc.VectorSubcoreMesh(core_axis_name="c", subcore_axis_name="s")
    fn = pl.kernel(
        _msg_sc, mesh=mesh,
        out_type=jax.ShapeDtypeStruct((NCORE, NP, AW), jnp.float32),
        scratch_types=[
            pltpu.VMEM((IBC * CH1,), jnp.int32),
            pltpu.VMEM((3, CH1), jnp.int32),
            pltpu.VMEM((2, CH1, YW), jnp.float32),
            pltpu.VMEM((2, CH1, HW), jnp.float32),
            pltpu.VMEM((3, CH1, AW), jnp.float32),
            pltpu.VMEM_SHARED((NP, AW), jnp.float32),
            pltpu.SemaphoreType.DMA((2,)),
            pltpu.SemaphoreType.DMA((2,)),
            pltpu.SemaphoreType.DMA((3,)),
        ],
    )
    return fn(y, h, src, dst)


def _node_body(msum_ref, x_ref, root_ref, eb_ref, gw_ref, asr_ref, adr_ref,
               tsrc_ref, tdst_ref, maxes_ref):
    i = pl.program_id(0)
    a = msum_ref[0] + msum_ref[1]
    deg = jnp.maximum(a[:, F:F + 1], 1.0)
    agg = a[:, :F] / deg
    x = jnp.maximum(
        agg + jnp.dot(x_ref[...], root_ref[...],
                      preferred_element_type=jnp.float32) + eb_ref[...], 0.0)
    xw = jnp.dot(x, gw_ref[...], preferred_element_type=jnp.float32)
    a_s = jnp.sum(xw * asr_ref[...], axis=1, keepdims=True)
    a_d = jnp.sum(xw * adr_ref[...], axis=1, keepdims=True)
    b = xw.shape[0]
    tsrc_ref[...] = jnp.concatenate(
        [xw, jnp.broadcast_to(a_s, (b, 16)),
         jnp.zeros((b, 128 - F - 16), jnp.float32)], axis=1)
    tdst_ref[...] = jnp.concatenate(
        [jnp.broadcast_to(a_d, (b, 16)),
         jnp.zeros((b, 112), jnp.float32)], axis=1)
    lane = lax.broadcasted_iota(jnp.int32, (1, 128), 1)
    cand = jnp.where(lane == 0, jnp.max(a_s),
                     jnp.where(lane == 1, jnp.max(a_d), -jnp.inf))

    @pl.when(i == 0)
    def _():
        maxes_ref[...] = jnp.full((1, 128), -jnp.inf, jnp.float32)
    maxes_ref[...] = jnp.maximum(maxes_ref[...], cand)


def _run_node(msum, x, root, eb, gw, asr, adr):
    blk = 1000
    grid = N // blk
    return pl.pallas_call(
        _node_body,
        grid=(grid,),
        in_specs=[
            pl.BlockSpec((NCORE, blk, AW), lambda i: (0, i, 0)),
            pl.BlockSpec((blk, D), lambda i: (i, 0)),
            pl.BlockSpec((D, F), lambda i: (0, 0)),
            pl.BlockSpec((1, F), lambda i: (0, 0)),
            pl.BlockSpec((F, F), lambda i: (0, 0)),
            pl.BlockSpec((1, F), lambda i: (0, 0)),
            pl.BlockSpec((1, F), lambda i: (0, 0)),
        ],
        out_specs=[
            pl.BlockSpec((blk, 128), lambda i: (i, 0)),
            pl.BlockSpec((blk, 128), lambda i: (i, 0)),
            pl.BlockSpec((1, 128), lambda i: (0, 0)),
        ],
        out_shape=[
            jax.ShapeDtypeStruct((N, 128), jnp.float32),
            jax.ShapeDtypeStruct((N, 128), jnp.float32),
            jax.ShapeDtypeStruct((1, 128), jnp.float32),
        ],
    )(msum, x, root, eb.reshape(1, F), gw, asr.reshape(1, F),
      adr.reshape(1, F))


def _gat_sc(tsrc_hbm, tdst_hbm, src_hbm, dst_hbm, m_hbm, out_hbm,
            src_v, dst_v, rx_v, rd_v, o_v, m_v, zbuf_v, accum_sh, sem):
    cid = lax.axis_index("c")
    sid = lax.axis_index("s")
    wid = cid * NSUB + sid

    def _zrow(r, _):
        for c in range(AW // 16):
            zbuf_v[r, pl.ds(c * 16, 16)] = jnp.zeros((16,), jnp.float32)
        return 0
    lax.fori_loop(0, ZR, _zrow, 0)
    for r in range(RPT // ZR):
        pltpu.sync_copy(zbuf_v, accum_sh.at[pl.ds(sid * RPT + r * ZR, ZR)])
    pltpu.sync_copy(m_hbm, m_v)
    plsc.subcore_barrier()

    base0 = wid * EPT

    def _chunk(ci, _):
        b = base0 + ci * CH2
        pltpu.sync_copy(src_hbm.at[pl.ds(b, CH2)], src_v)
        pltpu.sync_copy(dst_hbm.at[pl.ds(b, CH2)], dst_v)
        cp1 = pltpu.async_copy(tsrc_hbm.at[src_v], rx_v, sem)
        cp2 = pltpu.async_copy(tdst_hbm.at[dst_v], rd_v, sem)
        cp1.wait()
        cp2.wait()
        mvec = m_v[...]

        def _edge(e, _):
            s = rx_v[e, pl.ds(F, 16)] + rd_v[e, pl.ds(0, 16)]
            ee = jnp.maximum(s, 0.2 * s)
            w = jnp.exp(ee - mvec)
            for j in range(F // 16):
                o_v[e, pl.ds(j * 16, 16)] = w * rx_v[e, pl.ds(j * 16, 16)]
            o_v[e, pl.ds(F, 16)] = w
            for j in range(F // 16 + 1, AW // 16):
                o_v[e, pl.ds(j * 16, 16)] = jnp.zeros((16,), jnp.float32)
            return 0
        lax.fori_loop(0, CH2, _edge, 0)
        pltpu.sync_copy(o_v, accum_sh.at[dst_v], add=True)
        return 0
    lax.fori_loop(0, EPT // CH2, _chunk, 0)
    plsc.subcore_barrier()

    for r in range(RPT // ZR):
        pltpu.sync_copy(accum_sh.at[pl.ds(sid * RPT + r * ZR, ZR)], zbuf_v)
        pltpu.sync_copy(zbuf_v, out_hbm.at[cid, pl.ds(sid * RPT + r * ZR, ZR)])


def _run_gat(tsrc, tdst, src, dst, marr):
    mesh = plsc.VectorSubcoreMesh(core_axis_name="c", subcore_axis_name="s")
    fn = pl.kernel(
        _gat_sc, mesh=mesh,
        out_type=jax.ShapeDtypeStruct((NCORE, NP, AW), jnp.float32),
        scratch_types=[
            pltpu.VMEM((CH2,), jnp.int32),
            pltpu.VMEM((CH2,), jnp.int32),
            pltpu.VMEM((CH2, 128), jnp.float32),
            pltpu.VMEM((CH2, 128), jnp.float32),
            pltpu.VMEM((CH2, AW), jnp.float32),
            pltpu.VMEM((16,), jnp.float32),
            pltpu.VMEM((ZR, AW), jnp.float32),
            pltpu.VMEM_SHARED((NP, AW), jnp.float32),
            pltpu.SemaphoreType.DMA,
        ],
    )
    return fn(tsrc, tdst, src, dst, marr)


def _final_body(acc_ref, gb_ref, fw_ref, fb_ref, out_ref, psum_ref):
    i = pl.program_id(0)
    a = acc_ref[0] + acc_ref[1]
    den = jnp.maximum(a[:, F:F + 1], 1e-9)
    x2 = jnp.maximum(a[:, :F] / den + gb_ref[...], 0.0)

    @pl.when(i == 0)
    def _():
        psum_ref[...] = jnp.zeros((8, F), jnp.float32)
    psum_ref[0:1, :] = psum_ref[0:1, :] + jnp.sum(x2, axis=0, keepdims=True)
    pooled = psum_ref[0:1, :] * (1.0 / N)
    out_ref[...] = jnp.maximum(
        jnp.dot(pooled, fw_ref[...], preferred_element_type=jnp.float32)
        + fb_ref[...], 0.0)


def _run_final(acc, gb, fw, fb):
    blk = 1000
    grid = N // blk
    return pl.pallas_call(
        _final_body,
        grid=(grid,),
        in_specs=[
            pl.BlockSpec((NCORE, blk, AW), lambda i: (0, i, 0)),
            pl.BlockSpec((1, F), lambda i: (0, 0)),
            pl.BlockSpec((F, 32), lambda i: (0, 0)),
            pl.BlockSpec((1, 32), lambda i: (0, 0)),
        ],
        out_specs=pl.BlockSpec((1, 32), lambda i: (0, 0)),
        out_shape=jax.ShapeDtypeStruct((1, 32), jnp.float32),
        scratch_shapes=[pltpu.VMEM((8, F), jnp.float32)],
    )(acc, gb.reshape(1, F), fw, fb.reshape(1, 32))


def kernel(X_in, A_in, E_in, ecc_w1, ecc_b1, ecc_w2, ecc_b2, ecc_root,
           ecc_bias, gat_w, gat_bias, gat_attn_src, gat_attn_dst, fc_w, fc_b):
    src = A_in[0].astype(jnp.int32)
    dst = A_in[1].astype(jnp.int32)

    w2r = ecc_w2.reshape(KH, D, F).transpose(1, 0, 2).reshape(D, KH * F)
    w2ext = jnp.concatenate(
        [w2r, ecc_b2.reshape(D, F),
         jnp.zeros((D, YW - KTOT * F), jnp.float32)], axis=1)

    h = _make_h(E_in, ecc_w1, ecc_b1)
    y = _make_y(X_in, w2ext)
    msum = _run_msg(y, h, src, dst)
    tsrc, tdst, maxes = _run_node(msum, X_in, ecc_root, ecc_bias,
                                  gat_w, gat_attn_src, gat_attn_dst)
    m = jnp.maximum(maxes[0, 0] + maxes[0, 1], 0.0)
    marr = jnp.full((16,), m, jnp.float32)
    acc2 = _run_gat(tsrc, tdst, src, dst, marr)
    return _run_final(acc2, gat_bias, fc_w, fc_b)


# speedup vs baseline: 7.0440x; 1.0150x over previous
import functools
import jax
import jax.numpy as jnp
from jax import lax
from jax.experimental import pallas as pl
from jax.experimental.pallas import tpu as pltpu
from jax.experimental.pallas import tpu_sc as plsc

N = 10000
E = 320000
D = 128
F = 64
KH = 32
KTOT = KH + 1
HW = 528
YW = 2176
AW = 128
NSUB = 16
NCORE = 2
NTILE = NCORE * NSUB
EPT = E // NTILE
NP = 10112
RPT = NP // NSUB
CH1 = 8
ZR = 8
IBC = 125
CH2 = 40
IBC2 = 25


def _h_body(e_ref, w1_ref, b1_ref, rep_ref, out_ref):
    h = jnp.dot(e_ref[...], w1_ref[...], preferred_element_type=jnp.float32)
    h = jnp.maximum(h + b1_ref[...], 0.0)
    b = h.shape[0]
    hx = jnp.concatenate([h, jnp.ones((b, 1), jnp.float32)], axis=1)
    out_ref[...] = jnp.dot(hx, rep_ref[...], preferred_element_type=jnp.float32)


def _make_h(e_in, w1, b1):
    blk = 4000
    grid = E // blk
    rep = jnp.repeat(jnp.eye(KTOT, dtype=jnp.float32), 16, axis=1)
    return pl.pallas_call(
        _h_body,
        grid=(grid,),
        in_specs=[
            pl.BlockSpec((blk, 4), lambda i: (i, 0)),
            pl.BlockSpec((4, KH), lambda i: (0, 0)),
            pl.BlockSpec((1, KH), lambda i: (0, 0)),
            pl.BlockSpec((KTOT, HW), lambda i: (0, 0)),
        ],
        out_specs=pl.BlockSpec((blk, HW), lambda i: (i, 0)),
        out_shape=jax.ShapeDtypeStruct((E, HW), jnp.float32),
    )(e_in, w1, b1.reshape(1, KH), rep)


def _y_body(x_ref, w_ref, out_ref):
    out_ref[...] = jnp.dot(x_ref[...], w_ref[...],
                           preferred_element_type=jnp.float32)


def _make_y(x, w2ext):
    blk = 1000
    grid = N // blk
    return pl.pallas_call(
        _y_body,
        grid=(grid,),
        in_specs=[
            pl.BlockSpec((blk, D), lambda i: (i, 0)),
            pl.BlockSpec((D, YW), lambda i: (0, 0)),
        ],
        out_specs=pl.BlockSpec((blk, YW), lambda i: (i, 0)),
        out_shape=jax.ShapeDtypeStruct((N, YW), jnp.float32),
    )(x, w2ext)


def _msg_sc(y_hbm, h_hbm, src_hbm, dst_hbm, out_hbm,
            srcb_v, dst_v, rows_v, h_v, msg_v, accum_sh,
            sems_g, sems_m, sems_s):
    cid = lax.axis_index("c")
    sid = lax.axis_index("s")
    wid = cid * NSUB + sid

    def _zrow(r, _):
        for c in range(AW // 16):
            msg_v[0, r, pl.ds(c * 16, 16)] = jnp.zeros((16,), jnp.float32)
        return 0
    lax.fori_loop(0, ZR, _zrow, 0)
    for r in range(RPT // ZR):
        pltpu.sync_copy(msg_v.at[0], accum_sh.at[pl.ds(sid * RPT + r * ZR, ZR)])
    plsc.subcore_barrier()

    base0 = wid * EPT
    nchunks = EPT // CH1

    def _issue(ci):
        par = lax.rem(ci, 2)
        s3 = lax.rem(ci, 3)
        b = base0 + ci * CH1

        @pl.when(jnp.logical_and(lax.rem(ci, IBC) == 0, True))
        def _():
            pltpu.sync_copy(src_hbm.at[pl.ds(b, IBC * CH1)], srcb_v)

        @pl.when(ci >= 3)
        def _():
            pltpu.make_async_copy(msg_v.at[s3], accum_sh.at[pl.ds(0, CH1)],
                                  sems_s.at[s3]).wait()
        pltpu.async_copy(dst_hbm.at[pl.ds(b, CH1)], dst_v.at[s3],
                         sems_m.at[par])
        pltpu.async_copy(h_hbm.at[pl.ds(b, CH1)], h_v.at[par],
                         sems_m.at[par])
        off = lax.rem(ci, IBC) * CH1
        pltpu.async_copy(y_hbm.at[srcb_v.at[pl.ds(off, CH1)]],
                         rows_v.at[par], sems_g.at[par])

    _issue(0)

    def _chunk(ci, _):
        par = lax.rem(ci, 2)
        s3 = lax.rem(ci, 3)
        pltpu.make_async_copy(y_hbm.at[pl.ds(0, CH1)], rows_v.at[par],
                              sems_g.at[par]).wait()
        pltpu.make_async_copy(dst_hbm.at[pl.ds(0, CH1)], dst_v.at[s3],
                              sems_m.at[par]).wait()
        pltpu.make_async_copy(h_hbm.at[pl.ds(0, CH1)], h_v.at[par],
                              sems_m.at[par]).wait()

        @pl.when(ci + 1 < nchunks)
        def _():
            _issue(ci + 1)

        def _edge(e, _):
            acc = [jnp.zeros((16,), jnp.float32) for _ in range(F // 16)]
            for k in range(KTOT):
                hk = h_v[par, e, pl.ds(k * 16, 16)]
                for j in range(F // 16):
                    acc[j] = acc[j] + hk * rows_v[par, e,
                                                  pl.ds(k * F + j * 16, 16)]
            for j in range(F // 16):
                msg_v[s3, e, pl.ds(j * 16, 16)] = acc[j]
            msg_v[s3, e, pl.ds(F, 16)] = jnp.ones((16,), jnp.float32)
            for j in range(F // 16 + 1, AW // 16):
                msg_v[s3, e, pl.ds(j * 16, 16)] = jnp.zeros((16,), jnp.float32)
            return 0
        lax.fori_loop(0, CH1, _edge, 0)
        pltpu.async_copy(msg_v.at[s3], accum_sh.at[dst_v.at[s3]],
                         sems_s.at[s3], add=True)
        return 0
    lax.fori_loop(0, nchunks, _chunk, 0)
    for sl in range(3):
        pltpu.make_async_copy(msg_v.at[sl], accum_sh.at[pl.ds(0, CH1)],
                              sems_s.at[sl]).wait()
    plsc.subcore_barrier()

    pltpu.sync_copy(accum_sh.at[pl.ds(sid * RPT, RPT)],
                    out_hbm.at[cid, pl.ds(sid * RPT, RPT)])


def _run_msg(y, h, src, dst):
    mesh = plsc.VectorSubcoreMesh(core_axis_name="c", subcore_axis_name="s")
    fn = pl.kernel(
        _msg_sc, mesh=mesh,
        out_type=jax.ShapeDtypeStruct((NCORE, NP, AW), jnp.float32),
        scratch_types=[
            pltpu.VMEM((IBC * CH1,), jnp.int32),
            pltpu.VMEM((3, CH1), jnp.int32),
            pltpu.VMEM((2, CH1, YW), jnp.float32),
            pltpu.VMEM((2, CH1, HW), jnp.float32),
            pltpu.VMEM((3, CH1, AW), jnp.float32),
            pltpu.VMEM_SHARED((NP, AW), jnp.float32),
            pltpu.SemaphoreType.DMA((2,)),
            pltpu.SemaphoreType.DMA((2,)),
            pltpu.SemaphoreType.DMA((3,)),
        ],
    )
    return fn(y, h, src, dst)


def _node_body(msum_ref, x_ref, root_ref, eb_ref, gw_ref, asr_ref, adr_ref,
               tsrc_ref, tdst_ref, maxes_ref):
    i = pl.program_id(0)
    a = msum_ref[0] + msum_ref[1]
    deg = jnp.maximum(a[:, F:F + 1], 1.0)
    agg = a[:, :F] / deg
    x = jnp.maximum(
        agg + jnp.dot(x_ref[...], root_ref[...],
                      preferred_element_type=jnp.float32) + eb_ref[...], 0.0)
    xw = jnp.dot(x, gw_ref[...], preferred_element_type=jnp.float32)
    a_s = jnp.sum(xw * asr_ref[...], axis=1, keepdims=True)
    a_d = jnp.sum(xw * adr_ref[...], axis=1, keepdims=True)
    b = xw.shape[0]
    tsrc_ref[...] = jnp.concatenate(
        [xw, jnp.broadcast_to(a_s, (b, 16)),
         jnp.zeros((b, 128 - F - 16), jnp.float32)], axis=1)
    tdst_ref[...] = jnp.concatenate(
        [jnp.broadcast_to(a_d, (b, 16)),
         jnp.zeros((b, 112), jnp.float32)], axis=1)
    lane = lax.broadcasted_iota(jnp.int32, (1, 128), 1)
    cand = jnp.where(lane == 0, jnp.max(a_s),
                     jnp.where(lane == 1, jnp.max(a_d), -jnp.inf))

    @pl.when(i == 0)
    def _():
        maxes_ref[...] = jnp.full((1, 128), -jnp.inf, jnp.float32)
    maxes_ref[...] = jnp.maximum(maxes_ref[...], cand)


def _run_node(msum, x, root, eb, gw, asr, adr):
    blk = 1000
    grid = N // blk
    return pl.pallas_call(
        _node_body,
        grid=(grid,),
        in_specs=[
            pl.BlockSpec((NCORE, blk, AW), lambda i: (0, i, 0)),
            pl.BlockSpec((blk, D), lambda i: (i, 0)),
            pl.BlockSpec((D, F), lambda i: (0, 0)),
            pl.BlockSpec((1, F), lambda i: (0, 0)),
            pl.BlockSpec((F, F), lambda i: (0, 0)),
            pl.BlockSpec((1, F), lambda i: (0, 0)),
            pl.BlockSpec((1, F), lambda i: (0, 0)),
        ],
        out_specs=[
            pl.BlockSpec((blk, 128), lambda i: (i, 0)),
            pl.BlockSpec((blk, 128), lambda i: (i, 0)),
            pl.BlockSpec((1, 128), lambda i: (0, 0)),
        ],
        out_shape=[
            jax.ShapeDtypeStruct((N, 128), jnp.float32),
            jax.ShapeDtypeStruct((N, 128), jnp.float32),
            jax.ShapeDtypeStruct((1, 128), jnp.float32),
        ],
    )(msum, x, root, eb.reshape(1, F), gw, asr.reshape(1, F),
      adr.reshape(1, F))


def _gat_sc(tsrc_hbm, tdst_hbm, src_hbm, dst_hbm, m_hbm, out_hbm,
            srcb_v, dstb_v, dst_v, rx_v, rd_v, o_v, m_v, accum_sh,
            sems_g, sems_m, sems_s):
    cid = lax.axis_index("c")
    sid = lax.axis_index("s")
    wid = cid * NSUB + sid

    def _zrow(r, _):
        for c in range(AW // 16):
            o_v[0, r, pl.ds(c * 16, 16)] = jnp.zeros((16,), jnp.float32)
        return 0
    lax.fori_loop(0, ZR, _zrow, 0)
    for r in range(RPT // ZR):
        pltpu.sync_copy(o_v.at[0].at[pl.ds(0, ZR)],
                        accum_sh.at[pl.ds(sid * RPT + r * ZR, ZR)])
    pltpu.sync_copy(m_hbm, m_v)
    plsc.subcore_barrier()

    base0 = wid * EPT
    nchunks = EPT // CH2

    def _issue(ci):
        par = lax.rem(ci, 2)
        s3 = lax.rem(ci, 3)
        b = base0 + ci * CH2

        @pl.when(lax.rem(ci, IBC2) == 0)
        def _():
            pltpu.sync_copy(src_hbm.at[pl.ds(b, IBC2 * CH2)], srcb_v)
            pltpu.sync_copy(dst_hbm.at[pl.ds(b, IBC2 * CH2)], dstb_v)

        @pl.when(ci >= 3)
        def _():
            pltpu.make_async_copy(o_v.at[s3], accum_sh.at[pl.ds(0, CH2)],
                                  sems_s.at[s3]).wait()
        pltpu.async_copy(dst_hbm.at[pl.ds(b, CH2)], dst_v.at[s3],
                         sems_m.at[par])
        off = lax.rem(ci, IBC2) * CH2
        pltpu.async_copy(tsrc_hbm.at[srcb_v.at[pl.ds(off, CH2)]],
                         rx_v.at[par], sems_g.at[par])
        pltpu.async_copy(tdst_hbm.at[dstb_v.at[pl.ds(off, CH2)]],
                         rd_v.at[par], sems_g.at[par])

    _issue(0)

    def _chunk(ci, _):
        par = lax.rem(ci, 2)
        s3 = lax.rem(ci, 3)
        pltpu.make_async_copy(tsrc_hbm.at[pl.ds(0, CH2)], rx_v.at[par],
                              sems_g.at[par]).wait()
        pltpu.make_async_copy(tdst_hbm.at[pl.ds(0, CH2)], rd_v.at[par],
                              sems_g.at[par]).wait()
        pltpu.make_async_copy(dst_hbm.at[pl.ds(0, CH2)], dst_v.at[s3],
                              sems_m.at[par]).wait()

        @pl.when(ci + 1 < nchunks)
        def _():
            _issue(ci + 1)
        mvec = m_v[...]

        def _edge(e, _):
            sl = rx_v[par, e, pl.ds(F, 16)] + rd_v[par, e, pl.ds(0, 16)]
            ee = jnp.maximum(sl, 0.2 * sl)
            w = jnp.exp(ee - mvec)
            for j in range(F // 16):
                o_v[s3, e, pl.ds(j * 16, 16)] = w * rx_v[par, e,
                                                         pl.ds(j * 16, 16)]
            o_v[s3, e, pl.ds(F, 16)] = w
            for j in range(F // 16 + 1, AW // 16):
                o_v[s3, e, pl.ds(j * 16, 16)] = jnp.zeros((16,), jnp.float32)
            return 0
        lax.fori_loop(0, CH2, _edge, 0)
        pltpu.async_copy(o_v.at[s3], accum_sh.at[dst_v.at[s3]],
                         sems_s.at[s3], add=True)
        return 0
    lax.fori_loop(0, nchunks, _chunk, 0)
    for sl in range(3):
        pltpu.make_async_copy(o_v.at[sl], accum_sh.at[pl.ds(0, CH2)],
                              sems_s.at[sl]).wait()
    plsc.subcore_barrier()

    pltpu.sync_copy(accum_sh.at[pl.ds(sid * RPT, RPT)],
                    out_hbm.at[cid, pl.ds(sid * RPT, RPT)])


def _run_gat(tsrc, tdst, src, dst, marr):
    mesh = plsc.VectorSubcoreMesh(core_axis_name="c", subcore_axis_name="s")
    fn = pl.kernel(
        _gat_sc, mesh=mesh,
        out_type=jax.ShapeDtypeStruct((NCORE, NP, AW), jnp.float32),
        scratch_types=[
            pltpu.VMEM((IBC2 * CH2,), jnp.int32),
            pltpu.VMEM((IBC2 * CH2,), jnp.int32),
            pltpu.VMEM((3, CH2), jnp.int32),
            pltpu.VMEM((2, CH2, 128), jnp.float32),
            pltpu.VMEM((2, CH2, 128), jnp.float32),
            pltpu.VMEM((3, CH2, AW), jnp.float32),
            pltpu.VMEM((16,), jnp.float32),
            pltpu.VMEM_SHARED((NP, AW), jnp.float32),
            pltpu.SemaphoreType.DMA((2,)),
            pltpu.SemaphoreType.DMA((2,)),
            pltpu.SemaphoreType.DMA((3,)),
        ],
    )
    return fn(tsrc, tdst, src, dst, marr)


def _final_body(acc_ref, gb_ref, fw_ref, fb_ref, out_ref, psum_ref):
    i = pl.program_id(0)
    a = acc_ref[0] + acc_ref[1]
    den = jnp.maximum(a[:, F:F + 1], 1e-9)
    x2 = jnp.maximum(a[:, :F] / den + gb_ref[...], 0.0)

    @pl.when(i == 0)
    def _():
        psum_ref[...] = jnp.zeros((8, F), jnp.float32)
    psum_ref[0:1, :] = psum_ref[0:1, :] + jnp.sum(x2, axis=0, keepdims=True)
    pooled = psum_ref[0:1, :] * (1.0 / N)
    out_ref[...] = jnp.maximum(
        jnp.dot(pooled, fw_ref[...], preferred_element_type=jnp.float32)
        + fb_ref[...], 0.0)


def _run_final(acc, gb, fw, fb):
    blk = 1000
    grid = N // blk
    return pl.pallas_call(
        _final_body,
        grid=(grid,),
        in_specs=[
            pl.BlockSpec((NCORE, blk, AW), lambda i: (0, i, 0)),
            pl.BlockSpec((1, F), lambda i: (0, 0)),
            pl.BlockSpec((F, 32), lambda i: (0, 0)),
            pl.BlockSpec((1, 32), lambda i: (0, 0)),
        ],
        out_specs=pl.BlockSpec((1, 32), lambda i: (0, 0)),
        out_shape=jax.ShapeDtypeStruct((1, 32), jnp.float32),
        scratch_shapes=[pltpu.VMEM((8, F), jnp.float32)],
    )(acc, gb.reshape(1, F), fw, fb.reshape(1, 32))


def kernel(X_in, A_in, E_in, ecc_w1, ecc_b1, ecc_w2, ecc_b2, ecc_root,
           ecc_bias, gat_w, gat_bias, gat_attn_src, gat_attn_dst, fc_w, fc_b):
    src = A_in[0].astype(jnp.int32)
    dst = A_in[1].astype(jnp.int32)

    w2r = ecc_w2.reshape(KH, D, F).transpose(1, 0, 2).reshape(D, KH * F)
    w2ext = jnp.concatenate(
        [w2r, ecc_b2.reshape(D, F),
         jnp.zeros((D, YW - KTOT * F), jnp.float32)], axis=1)

    h = _make_h(E_in, ecc_w1, ecc_b1)
    y = _make_y(X_in, w2ext)
    msum = _run_msg(y, h, src, dst)
    tsrc, tdst, maxes = _run_node(msum, X_in, ecc_root, ecc_bias,
                                  gat_w, gat_attn_src, gat_attn_dst)
    m = jnp.maximum(maxes[0, 0] + maxes[0, 1], 0.0)
    marr = jnp.full((16,), m, jnp.float32)
    acc2 = _run_gat(tsrc, tdst, src, dst, marr)
    return _run_final(acc2, gat_bias, fc_w, fc_b)


# speedup vs baseline: 7.0460x; 1.0003x over previous
import functools
import jax
import jax.numpy as jnp
from jax import lax
from jax.experimental import pallas as pl
from jax.experimental.pallas import tpu as pltpu
from jax.experimental.pallas import tpu_sc as plsc

N = 10000
E = 320000
D = 128
F = 64
KH = 32
KTOT = KH + 1
KTOT2 = 36
HW = 576
YW = 2176
AW = 128
NSUB = 16
NCORE = 2
NTILE = NCORE * NSUB
EPT = E // NTILE
NP = 10112
RPT = NP // NSUB
CH1 = 8
ZR = 8
IBC = 125
CH2 = 40
IBC2 = 25


def _h_body(e_ref, w1_ref, b1_ref, rep_ref, out_ref):
    h = jnp.dot(e_ref[...], w1_ref[...], preferred_element_type=jnp.float32)
    h = jnp.maximum(h + b1_ref[...], 0.0)
    b = h.shape[0]
    hx = jnp.concatenate([h, jnp.ones((b, 1), jnp.float32),
                          jnp.zeros((b, 3), jnp.float32)], axis=1)
    out_ref[...] = jnp.dot(hx, rep_ref[...], preferred_element_type=jnp.float32)


def _make_h(e_in, w1, b1):
    blk = 4000
    grid = E // blk
    rep = jnp.repeat(jnp.eye(KTOT2, dtype=jnp.float32), 16, axis=1)
    return pl.pallas_call(
        _h_body,
        grid=(grid,),
        in_specs=[
            pl.BlockSpec((blk, 4), lambda i: (i, 0)),
            pl.BlockSpec((4, KH), lambda i: (0, 0)),
            pl.BlockSpec((1, KH), lambda i: (0, 0)),
            pl.BlockSpec((KTOT2, HW), lambda i: (0, 0)),
        ],
        out_specs=pl.BlockSpec((blk, HW), lambda i: (i, 0)),
        out_shape=jax.ShapeDtypeStruct((E, HW), jnp.float32),
    )(e_in, w1, b1.reshape(1, KH), rep)


def _y_body(x_ref, w_ref, out_ref):
    out_ref[...] = jnp.dot(x_ref[...], w_ref[...],
                           preferred_element_type=jnp.float32)


def _make_y(x, w2ext):
    blk = 1000
    grid = N // blk
    return pl.pallas_call(
        _y_body,
        grid=(grid,),
        in_specs=[
            pl.BlockSpec((blk, D), lambda i: (i, 0)),
            pl.BlockSpec((D, YW), lambda i: (0, 0)),
        ],
        out_specs=pl.BlockSpec((blk, YW), lambda i: (i, 0)),
        out_shape=jax.ShapeDtypeStruct((N, YW), jnp.float32),
    )(x, w2ext)


def _msg_sc(y_hbm, h_hbm, src_hbm, dst_hbm, out_hbm,
            srcb_v, dst_v, rows_v, h_v, msg_v, accum_sh,
            sems_g, sems_m, sems_s):
    cid = lax.axis_index("c")
    sid = lax.axis_index("s")
    wid = cid * NSUB + sid

    def _zrow(r, _):
        for c in range(AW // 16):
            msg_v[0, r, pl.ds(c * 16, 16)] = jnp.zeros((16,), jnp.float32)
        return 0
    lax.fori_loop(0, ZR, _zrow, 0)
    for r in range(RPT // ZR):
        pltpu.sync_copy(msg_v.at[0], accum_sh.at[pl.ds(sid * RPT + r * ZR, ZR)])
    plsc.subcore_barrier()

    base0 = wid * EPT
    nchunks = EPT // CH1

    def _issue(ci):
        par = lax.rem(ci, 2)
        s3 = lax.rem(ci, 3)
        b = base0 + ci * CH1

        @pl.when(jnp.logical_and(lax.rem(ci, IBC) == 0, True))
        def _():
            pltpu.sync_copy(src_hbm.at[pl.ds(b, IBC * CH1)], srcb_v)

        @pl.when(ci >= 3)
        def _():
            pltpu.make_async_copy(msg_v.at[s3], accum_sh.at[pl.ds(0, CH1)],
                                  sems_s.at[s3]).wait()
        pltpu.async_copy(dst_hbm.at[pl.ds(b, CH1)], dst_v.at[s3],
                         sems_m.at[par])
        pltpu.async_copy(h_hbm.at[pl.ds(b, CH1)], h_v.at[par],
                         sems_m.at[par])
        off = lax.rem(ci, IBC) * CH1
        pltpu.async_copy(y_hbm.at[srcb_v.at[pl.ds(off, CH1)]],
                         rows_v.at[par], sems_g.at[par])

    _issue(0)

    def _chunk(ci, _):
        par = lax.rem(ci, 2)
        s3 = lax.rem(ci, 3)
        pltpu.make_async_copy(y_hbm.at[pl.ds(0, CH1)], rows_v.at[par],
                              sems_g.at[par]).wait()
        pltpu.make_async_copy(dst_hbm.at[pl.ds(0, CH1)], dst_v.at[s3],
                              sems_m.at[par]).wait()
        pltpu.make_async_copy(h_hbm.at[pl.ds(0, CH1)], h_v.at[par],
                              sems_m.at[par]).wait()

        @pl.when(ci + 1 < nchunks)
        def _():
            _issue(ci + 1)

        def _edge(e, _):
            acc = [jnp.zeros((16,), jnp.float32) for _ in range(F // 16)]
            for k in range(KTOT):
                hk = h_v[par, e, pl.ds(k * 16, 16)]
                for j in range(F // 16):
                    acc[j] = acc[j] + hk * rows_v[par, e,
                                                  pl.ds(k * F + j * 16, 16)]
            for j in range(F // 16):
                msg_v[s3, e, pl.ds(j * 16, 16)] = acc[j]
            msg_v[s3, e, pl.ds(F, 16)] = jnp.ones((16,), jnp.float32)
            for j in range(F // 16 + 1, AW // 16):
                msg_v[s3, e, pl.ds(j * 16, 16)] = jnp.zeros((16,), jnp.float32)
            return 0
        lax.fori_loop(0, CH1, _edge, 0)
        pltpu.async_copy(msg_v.at[s3], accum_sh.at[dst_v.at[s3]],
                         sems_s.at[s3], add=True)
        return 0
    lax.fori_loop(0, nchunks, _chunk, 0)
    for sl in range(3):
        pltpu.make_async_copy(msg_v.at[sl], accum_sh.at[pl.ds(0, CH1)],
                              sems_s.at[sl]).wait()
    plsc.subcore_barrier()

    pltpu.sync_copy(accum_sh.at[pl.ds(sid * RPT, RPT)],
                    out_hbm.at[cid, pl.ds(sid * RPT, RPT)])


def _run_msg(y, h, src, dst):
    mesh = plsc.VectorSubcoreMesh(core_axis_name="c", subcore_axis_name="s")
    fn = pl.kernel(
        _msg_sc, mesh=mesh,
        out_type=jax.ShapeDtypeStruct((NCORE, NP, AW), jnp.float32),
        scratch_types=[
            pltpu.VMEM((IBC * CH1,), jnp.int32),
            pltpu.VMEM((3, CH1), jnp.int32),
            pltpu.VMEM((2, CH1, YW), jnp.float32),
            pltpu.VMEM((2, CH1, HW), jnp.float32),
            pltpu.VMEM((3, CH1, AW), jnp.float32),
            pltpu.VMEM_SHARED((NP, AW), jnp.float32),
            pltpu.SemaphoreType.DMA((2,)),
            pltpu.SemaphoreType.DMA((2,)),
            pltpu.SemaphoreType.DMA((3,)),
        ],
    )
    return fn(y, h, src, dst)


def _node_body(msum_ref, x_ref, root_ref, eb_ref, gw_ref, asr_ref, adr_ref,
               tsrc_ref, tdst_ref, maxes_ref):
    i = pl.program_id(0)
    a = msum_ref[0] + msum_ref[1]
    deg = jnp.maximum(a[:, F:F + 1], 1.0)
    agg = a[:, :F] / deg
    x = jnp.maximum(
        agg + jnp.dot(x_ref[...], root_ref[...],
                      preferred_element_type=jnp.float32) + eb_ref[...], 0.0)
    xw = jnp.dot(x, gw_ref[...], preferred_element_type=jnp.float32)
    a_s = jnp.sum(xw * asr_ref[...], axis=1, keepdims=True)
    a_d = jnp.sum(xw * adr_ref[...], axis=1, keepdims=True)
    b = xw.shape[0]
    tsrc_ref[...] = jnp.concatenate(
        [xw, jnp.broadcast_to(a_s, (b, 16)),
         jnp.zeros((b, 128 - F - 16), jnp.float32)], axis=1)
    tdst_ref[...] = jnp.concatenate(
        [jnp.broadcast_to(a_d, (b, 16)),
         jnp.zeros((b, 112), jnp.float32)], axis=1)
    lane = lax.broadcasted_iota(jnp.int32, (1, 128), 1)
    cand = jnp.where(lane == 0, jnp.max(a_s),
                     jnp.where(lane == 1, jnp.max(a_d), -jnp.inf))

    @pl.when(i == 0)
    def _():
        maxes_ref[...] = jnp.full((1, 128), -jnp.inf, jnp.float32)
    maxes_ref[...] = jnp.maximum(maxes_ref[...], cand)


def _run_node(msum, x, root, eb, gw, asr, adr):
    blk = 1000
    grid = N // blk
    return pl.pallas_call(
        _node_body,
        grid=(grid,),
        in_specs=[
            pl.BlockSpec((NCORE, blk, AW), lambda i: (0, i, 0)),
            pl.BlockSpec((blk, D), lambda i: (i, 0)),
            pl.BlockSpec((D, F), lambda i: (0, 0)),
            pl.BlockSpec((1, F), lambda i: (0, 0)),
            pl.BlockSpec((F, F), lambda i: (0, 0)),
            pl.BlockSpec((1, F), lambda i: (0, 0)),
            pl.BlockSpec((1, F), lambda i: (0, 0)),
        ],
        out_specs=[
            pl.BlockSpec((blk, 128), lambda i: (i, 0)),
            pl.BlockSpec((blk, 128), lambda i: (i, 0)),
            pl.BlockSpec((1, 128), lambda i: (0, 0)),
        ],
        out_shape=[
            jax.ShapeDtypeStruct((N, 128), jnp.float32),
            jax.ShapeDtypeStruct((N, 128), jnp.float32),
            jax.ShapeDtypeStruct((1, 128), jnp.float32),
        ],
    )(msum, x, root, eb.reshape(1, F), gw, asr.reshape(1, F),
      adr.reshape(1, F))


def _gat_sc(tsrc_hbm, tdst_hbm, src_hbm, dst_hbm, m_hbm, out_hbm,
            srcb_v, dstb_v, dst_v, rx_v, rd_v, o_v, m_v, accum_sh,
            sems_g, sems_m, sems_s):
    cid = lax.axis_index("c")
    sid = lax.axis_index("s")
    wid = cid * NSUB + sid

    def _zrow(r, _):
        for c in range(AW // 16):
            o_v[0, r, pl.ds(c * 16, 16)] = jnp.zeros((16,), jnp.float32)
        return 0
    lax.fori_loop(0, ZR, _zrow, 0)
    for r in range(RPT // ZR):
        pltpu.sync_copy(o_v.at[0].at[pl.ds(0, ZR)],
                        accum_sh.at[pl.ds(sid * RPT + r * ZR, ZR)])
    pltpu.sync_copy(m_hbm, m_v)
    plsc.subcore_barrier()

    base0 = wid * EPT
    nchunks = EPT // CH2

    def _issue(ci):
        par = lax.rem(ci, 2)
        s3 = lax.rem(ci, 3)
        b = base0 + ci * CH2

        @pl.when(lax.rem(ci, IBC2) == 0)
        def _():
            pltpu.sync_copy(src_hbm.at[pl.ds(b, IBC2 * CH2)], srcb_v)
            pltpu.sync_copy(dst_hbm.at[pl.ds(b, IBC2 * CH2)], dstb_v)

        @pl.when(ci >= 3)
        def _():
            pltpu.make_async_copy(o_v.at[s3], accum_sh.at[pl.ds(0, CH2)],
                                  sems_s.at[s3]).wait()
        pltpu.async_copy(dst_hbm.at[pl.ds(b, CH2)], dst_v.at[s3],
                         sems_m.at[par])
        off = lax.rem(ci, IBC2) * CH2
        pltpu.async_copy(tsrc_hbm.at[srcb_v.at[pl.ds(off, CH2)]],
                         rx_v.at[par], sems_g.at[par])
        pltpu.async_copy(tdst_hbm.at[dstb_v.at[pl.ds(off, CH2)]],
                         rd_v.at[par], sems_g.at[par])

    _issue(0)

    def _chunk(ci, _):
        par = lax.rem(ci, 2)
        s3 = lax.rem(ci, 3)
        pltpu.make_async_copy(tsrc_hbm.at[pl.ds(0, CH2)], rx_v.at[par],
                              sems_g.at[par]).wait()
        pltpu.make_async_copy(tdst_hbm.at[pl.ds(0, CH2)], rd_v.at[par],
                              sems_g.at[par]).wait()
        pltpu.make_async_copy(dst_hbm.at[pl.ds(0, CH2)], dst_v.at[s3],
                              sems_m.at[par]).wait()

        @pl.when(ci + 1 < nchunks)
        def _():
            _issue(ci + 1)
        mvec = m_v[...]

        def _edge(e, _):
            sl = rx_v[par, e, pl.ds(F, 16)] + rd_v[par, e, pl.ds(0, 16)]
            ee = jnp.maximum(sl, 0.2 * sl)
            w = jnp.exp(ee - mvec)
            for j in range(F // 16):
                o_v[s3, e, pl.ds(j * 16, 16)] = w * rx_v[par, e,
                                                         pl.ds(j * 16, 16)]
            o_v[s3, e, pl.ds(F, 16)] = w
            for j in range(F // 16 + 1, AW // 16):
                o_v[s3, e, pl.ds(j * 16, 16)] = jnp.zeros((16,), jnp.float32)
            return 0
        lax.fori_loop(0, CH2, _edge, 0)
        pltpu.async_copy(o_v.at[s3], accum_sh.at[dst_v.at[s3]],
                         sems_s.at[s3], add=True)
        return 0
    lax.fori_loop(0, nchunks, _chunk, 0)
    for sl in range(3):
        pltpu.make_async_copy(o_v.at[sl], accum_sh.at[pl.ds(0, CH2)],
                              sems_s.at[sl]).wait()
    plsc.subcore_barrier()

    pltpu.sync_copy(accum_sh.at[pl.ds(sid * RPT, RPT)],
                    out_hbm.at[cid, pl.ds(sid * RPT, RPT)])


def _run_gat(tsrc, tdst, src, dst, marr):
    mesh = plsc.VectorSubcoreMesh(core_axis_name="c", subcore_axis_name="s")
    fn = pl.kernel(
        _gat_sc, mesh=mesh,
        out_type=jax.ShapeDtypeStruct((NCORE, NP, AW), jnp.float32),
        scratch_types=[
            pltpu.VMEM((IBC2 * CH2,), jnp.int32),
            pltpu.VMEM((IBC2 * CH2,), jnp.int32),
            pltpu.VMEM((3, CH2), jnp.int32),
            pltpu.VMEM((2, CH2, 128), jnp.float32),
            pltpu.VMEM((2, CH2, 128), jnp.float32),
            pltpu.VMEM((3, CH2, AW), jnp.float32),
            pltpu.VMEM((16,), jnp.float32),
            pltpu.VMEM_SHARED((NP, AW), jnp.float32),
            pltpu.SemaphoreType.DMA((2,)),
            pltpu.SemaphoreType.DMA((2,)),
            pltpu.SemaphoreType.DMA((3,)),
        ],
    )
    return fn(tsrc, tdst, src, dst, marr)


def _final_body(acc_ref, gb_ref, fw_ref, fb_ref, out_ref, psum_ref):
    i = pl.program_id(0)
    a = acc_ref[0] + acc_ref[1]
    den = jnp.maximum(a[:, F:F + 1], 1e-9)
    x2 = jnp.maximum(a[:, :F] / den + gb_ref[...], 0.0)

    @pl.when(i == 0)
    def _():
        psum_ref[...] = jnp.zeros((8, F), jnp.float32)
    psum_ref[0:1, :] = psum_ref[0:1, :] + jnp.sum(x2, axis=0, keepdims=True)
    pooled = psum_ref[0:1, :] * (1.0 / N)
    out_ref[...] = jnp.maximum(
        jnp.dot(pooled, fw_ref[...], preferred_element_type=jnp.float32)
        + fb_ref[...], 0.0)


def _run_final(acc, gb, fw, fb):
    blk = 1000
    grid = N // blk
    return pl.pallas_call(
        _final_body,
        grid=(grid,),
        in_specs=[
            pl.BlockSpec((NCORE, blk, AW), lambda i: (0, i, 0)),
            pl.BlockSpec((1, F), lambda i: (0, 0)),
            pl.BlockSpec((F, 32), lambda i: (0, 0)),
            pl.BlockSpec((1, 32), lambda i: (0, 0)),
        ],
        out_specs=pl.BlockSpec((1, 32), lambda i: (0, 0)),
        out_shape=jax.ShapeDtypeStruct((1, 32), jnp.float32),
        scratch_shapes=[pltpu.VMEM((8, F), jnp.float32)],
    )(acc, gb.reshape(1, F), fw, fb.reshape(1, 32))


def kernel(X_in, A_in, E_in, ecc_w1, ecc_b1, ecc_w2, ecc_b2, ecc_root,
           ecc_bias, gat_w, gat_bias, gat_attn_src, gat_attn_dst, fc_w, fc_b):
    src = A_in[0].astype(jnp.int32)
    dst = A_in[1].astype(jnp.int32)

    w2r = ecc_w2.reshape(KH, D, F).transpose(1, 0, 2).reshape(D, KH * F)
    w2ext = jnp.concatenate(
        [w2r, ecc_b2.reshape(D, F),
         jnp.zeros((D, YW - KTOT * F), jnp.float32)], axis=1)


    h = _make_h(E_in, ecc_w1, ecc_b1)
    y = _make_y(X_in, w2ext)
    msum = _run_msg(y, h, src, dst)
    tsrc, tdst, maxes = _run_node(msum, X_in, ecc_root, ecc_bias,
                                  gat_w, gat_attn_src, gat_attn_dst)
    m = jnp.maximum(maxes[0, 0] + maxes[0, 1], 0.0)
    marr = jnp.full((16,), m, jnp.float32)
    acc2 = _run_gat(tsrc, tdst, src, dst, marr)
    return _run_final(acc2, gat_bias, fc_w, fc_b)
